# baseline (speedup 1.0000x reference)
import functools
import jax
import jax.numpy as jnp
from jax import lax
from jax.experimental import pallas as pl
from jax.experimental.pallas import tpu as pltpu
from jax.experimental.pallas import tpu_sc as plsc

N = 10000
E = 320000
IN_CH = 128
HID = 16
HEADS = 8
OUT_CH = 32
EDGE_DIM = 16
NEG = 0.2

NCORE = 2
NSUB = 16
NWORK = NCORE * NSUB
EPW = E // NWORK
WIN = 40
NWIN = EPW // WIN


def _mm(x, w, b, act=None, block=1000):
    n, fi = x.shape
    fo = w.shape[1]

    def kern(x_ref, w_ref, b_ref, o_ref):
        y = jnp.dot(x_ref[...], w_ref[...],
                    preferred_element_type=jnp.float32) + b_ref[...]
        if act == "elu":
            y = jnp.where(y > 0, y, jnp.exp(y) - 1.0)
        o_ref[...] = y

    return pl.pallas_call(
        kern,
        grid=(n // block,),
        in_specs=[
            pl.BlockSpec((block, fi), lambda i: (i, 0)),
            pl.BlockSpec((fi, fo), lambda i: (0, 0)),
            pl.BlockSpec((1, fo), lambda i: (0, 0)),
        ],
        out_specs=pl.BlockSpec((block, fo), lambda i: (i, 0)),
        out_shape=jax.ShapeDtypeStruct((n, fo), jnp.float32),
    )(x, w, b.reshape(1, fo))


def _masked_sum(edge_attr, maskf):
    BE = 8000
    G = E // BE

    def kern(a_ref, m_ref, o_ref):
        s = jnp.sum(a_ref[...] * m_ref[...], axis=0, keepdims=True)
        c = jnp.sum(m_ref[...]).reshape(1, 1)
        row = jnp.concatenate(
            [s, c, jnp.zeros((1, 128 - EDGE_DIM - 1), jnp.float32)], axis=1)
        o_ref[...] = jnp.concatenate(
            [row, jnp.zeros((7, 128), jnp.float32)], axis=0)

    return pl.pallas_call(
        kern,
        grid=(G,),
        in_specs=[
            pl.BlockSpec((BE, EDGE_DIM), lambda i: (i, 0)),
            pl.BlockSpec((BE, 1), lambda i: (i, 0)),
        ],
        out_specs=pl.BlockSpec((8, 128), lambda i: (i, 0)),
        out_shape=jax.ShapeDtypeStruct((8 * G, 128), jnp.float32),
    )(edge_attr, maskf)


def _sc_edge_pass(src, dst, xl, xr, e_all, att_flat, zeros, hc, H):
    mesh = plsc.VectorSubcoreMesh(core_axis_name="c", subcore_axis_name="s",
                                  num_cores=NCORE, num_subcores=NSUB)
    ND = N // 16
    NDP = ND + 7

    def body(src_hbm, dst_hbm, xl_hbm, xr_hbm, e_hbm, att_hbm, zero_hbm,
             out_hbm, den_hbm,
             srcvA, dstvA, dstpA, dst8vA, glvA, grvA, evA, maskvA,
             semLA, semG1A, semG2A,
             srcvB, dstvB, dstpB, dst8vB, glvB, grvB, evB, maskvB,
             semLB, semG1B, semG2B,
             obuf2, attv, exbuf, acc_sh, den_sh):
        cid = lax.axis_index("c")
        sid = lax.axis_index("s")
        SETS = (
            (srcvA, dstvA, dstpA, dst8vA, glvA, grvA, evA, maskvA,
             semLA, semG1A, semG2A),
            (srcvB, dstvB, dstpB, dst8vB, glvB, grvB, evB, maskvB,
             semLB, semG1B, semG2B),
        )

        @pl.when(sid < 10)
        def _():
            pltpu.sync_copy(zero_hbm.at[pl.ds(sid * 1000, 1000)],
                            acc_sh.at[pl.ds(sid * 1000, 1000)])

        @pl.when(sid == 0)
        def _():
            pltpu.sync_copy(zero_hbm.at[pl.ds(0, NDP)], den_sh)

        pltpu.sync_copy(att_hbm, attv)
        zero16v = jnp.zeros((16,), jnp.float32)

        def orow(r, c):
            for g in range(8):
                obuf2[r, pl.ds(g * 16, 16)] = zero16v
            return c

        lax.fori_loop(0, WIN, orow, 0, unroll=False)
        plsc.subcore_barrier()

        att_regs = [attv[pl.ds(h * 16, 16)] for h in range(H)]
        lanes = lax.iota(jnp.int32, 16)
        zero16 = jnp.zeros((16,), jnp.float32)
        base0 = (cid * NSUB + sid) * EPW

        gdn = lax.GatherDimensionNumbers(
            offset_dims=(), collapsed_slice_dims=(0,), start_index_map=(0,))
        bfly_idx = [(jnp.arange(16, dtype=jnp.int32) ^ sh)[:, None]
                    for sh in (1, 2, 4, 8)]

        def lanesum(v):
            for idx in bfly_idx:
                v = v + lax.gather(v, idx, gdn, (1,), unique_indices=True,
                                   mode=lax.GatherScatterMode.PROMISE_IN_BOUNDS)
            return v

        rot8_idx = ((jnp.arange(16, dtype=jnp.int32) - 8) & 15)[:, None]

        def rot8(v):
            return lax.gather(v, rot8_idx, gdn, (1,), unique_indices=True,
                              mode=lax.GatherScatterMode.PROMISE_IN_BOUNDS)

        def prefetch(S, base):
            (srcv, dstv, dstp, dst8v, glv, grv, ev, maskv,
             semL, semG1, semG2) = S
            d1 = pltpu.async_copy(src_hbm.at[pl.ds(base, WIN)], srcv, semL)
            d2 = pltpu.async_copy(dst_hbm.at[pl.ds(base, WIN)], dstv, semL)
            d3 = pltpu.async_copy(dst_hbm.at[pl.ds(base, WIN)],
                                  dstp.at[pl.ds(0, WIN)], semL)
            d4 = pltpu.async_copy(e_hbm.at[pl.ds(base, WIN)], ev, semL)
            d1.wait()
            d2.wait()
            d3.wait()
            d4.wait()
            for g0 in (0, 16, WIN - 16):
                sv = srcv[pl.ds(g0, 16)]
                dv = dstv[pl.ds(g0, 16)]
                maskv[pl.ds(g0, 16)] = jnp.where(sv != dv, 1.0, 0.0)
                dst8v[pl.ds(g0, 16)] = lax.shift_right_logical(dv, 4)
            pltpu.async_copy(xl_hbm.at[srcv], glv, semG1)
            pltpu.async_copy(xr_hbm.at[dstv], grv, semG2)

        def do_window(S):
            (srcv, dstv, dstp, dst8v, glv, grv, ev, maskv,
             semL, semG1, semG2) = S
            pltpu.make_async_copy(xl_hbm.at[srcv], glv, semG1).wait()
            pltpu.make_async_copy(xr_hbm.at[dstv], grv, semG2).wait()

            def edge1(i, c):
                maskf = maskv[pl.ds(i, 16)][0]
                gls = []
                av = zero16
                for h in range(H):
                    gl_h = glv[i, pl.ds(h * 16, 16)]
                    gr_h = grv[i, pl.ds(h * 16, 16)]
                    e_h = ev[i, pl.ds(h * 16, 16)]
                    m = gl_h + gr_h + e_h
                    m = jnp.maximum(m, NEG * m)
                    av = jnp.where(lanes == h, lanesum(m * att_regs[h]), av)
                    gls.append(gl_h)
                exv = jnp.exp(av) * maskf
                exbuf[i, pl.ds(0, 16)] = exv
                for h in range(H):
                    glv[i, pl.ds(h * 16, 16)] = gls[h] * exv[h]
                return c

            def edge2(i, c):
                exv = exbuf[i, pl.ds(0, 16)]
                d_i = dstp[pl.ds(i, 16)][0]
                k = d_i & 15
                if H == 8:
                    grp = lax.shift_right_logical(k, 1)
                    hi = (k & 1) == 1
                    row16 = jnp.where(hi,
                                      jnp.where(lanes >= 8, rot8(exv), 0.0),
                                      jnp.where(lanes < 8, exv, 0.0))
                    for g4 in range(8):
                        obuf2[i, pl.ds(g4 * 16, 16)] = jnp.where(
                            grp == g4, row16, 0.0)
                else:
                    obuf2[i, pl.ds(0, 16)] = jnp.where(lanes == k, exv[0],
                                                       0.0)
                return c

            lax.fori_loop(0, WIN, edge1, 0, unroll=False)
            lax.fori_loop(0, WIN, edge2, 0, unroll=False)
            pltpu.sync_copy(glv, acc_sh.at[dstv], add=True)
            pltpu.sync_copy(obuf2, den_sh.at[dst8v], add=True)

        prefetch(SETS[0], base0)
        prefetch(SETS[1], base0 + WIN)

        def superstep(ks, carry):
            for off in (0, 1):
                S = SETS[off]
                do_window(S)

                @pl.when(ks < NWIN // 2 - 1)
                def _():
                    prefetch(S, base0 + (2 * ks + off + 2) * WIN)
            return carry

        lax.fori_loop(0, NWIN // 2, superstep, 0, unroll=False)

        plsc.subcore_barrier()

        @pl.when(sid < 10)
        def _():
            pltpu.sync_copy(acc_sh.at[pl.ds(sid * 1000, 1000)],
                            out_hbm.at[pl.ds(cid * N + sid * 1000, 1000)])

        @pl.when(sid == 0)
        def _():
            pltpu.sync_copy(den_sh, den_hbm.at[pl.ds(cid * NDP, NDP)])

    kern = pl.kernel(
        body,
        out_type=(
            jax.ShapeDtypeStruct((NCORE * N, 128), jnp.float32),
            jax.ShapeDtypeStruct((NCORE * NDP, 128), jnp.float32),
        ),
        mesh=mesh,
        scratch_types=(
            [
                pltpu.VMEM((WIN,), jnp.int32),
                pltpu.VMEM((WIN,), jnp.int32),
                pltpu.VMEM((WIN + 16,), jnp.int32),
                pltpu.VMEM((WIN,), jnp.int32),
                pltpu.VMEM((WIN, 128), jnp.float32),
                pltpu.VMEM((WIN, 128), jnp.float32),
                pltpu.VMEM((WIN, hc), jnp.float32),
                pltpu.VMEM((WIN + 16,), jnp.float32),
                pltpu.SemaphoreType.DMA,
                pltpu.SemaphoreType.DMA,
                pltpu.SemaphoreType.DMA,
            ] * 2
            + [
                pltpu.VMEM((WIN, 128), jnp.float32),
                pltpu.VMEM((hc,), jnp.float32),
                pltpu.VMEM((WIN, 16), jnp.float32),
                pltpu.VMEM_SHARED((N, 128), jnp.float32),
                pltpu.VMEM_SHARED((NDP, 128), jnp.float32),
            ]
        ),
    )
    return kern(src, dst, xl, xr, e_all, att_flat, zeros)


def _combine(acc0, acc1, dall, xl, xr, skip, mean_attr, We, att_row, bias,
             g, bnb, S, ST, hc, H, out_dim, do_elu):
    B = 1000

    def kern(a0, a1, dl, xl_r, xr_r, sk, ma, we, at, bi, gg, bb, s_r, st_r,
             o_ref):
        num = (a0[...] + a1[...])[:, :hc]
        den = jnp.sum(dl[...], axis=0)
        e_self = jnp.dot(ma[...], we[...],
                         preferred_element_type=jnp.float32)
        ms = xl_r[...] + xr_r[...] + e_self
        ms = jnp.maximum(ms, NEG * ms)
        alpha_s = jnp.dot(ms * at[...], s_r[...],
                          preferred_element_type=jnp.float32)
        exs = jnp.exp(alpha_s)
        rep = jnp.dot(exs, st_r[...], preferred_element_type=jnp.float32)
        num = num + rep * xl_r[...]
        denf = jnp.dot(den + exs, st_r[...],
                       preferred_element_type=jnp.float32) + 1e-16
        out = num / denf
        out = out + bi[...] + sk[...]
        mu = jnp.mean(out, axis=-1, keepdims=True)
        var = jnp.mean((out - mu) ** 2, axis=-1, keepdims=True)
        out = (out - mu) / jnp.sqrt(var + 1e-5) * gg[...] + bb[...]
        if do_elu:
            out = jnp.where(out > 0, out, jnp.exp(out) - 1.0)
        o_ref[...] = out

    return pl.pallas_call(
        kern,
        grid=(N // B,),
        in_specs=[
            pl.BlockSpec((B, 128), lambda i: (i, 0)),
            pl.BlockSpec((B, 128), lambda i: (i, 0)),
            pl.BlockSpec((NCORE, B, H), lambda i: (0, i, 0)),
            pl.BlockSpec((B, hc), lambda i: (i, 0)),
            pl.BlockSpec((B, hc), lambda i: (i, 0)),
            pl.BlockSpec((B, out_dim), lambda i: (i, 0)),
            pl.BlockSpec((1, EDGE_DIM), lambda i: (0, 0)),
            pl.BlockSpec((EDGE_DIM, hc), lambda i: (0, 0)),
            pl.BlockSpec((1, hc), lambda i: (0, 0)),
            pl.BlockSpec((1, out_dim), lambda i: (0, 0)),
            pl.BlockSpec((1, out_dim), lambda i: (0, 0)),
            pl.BlockSpec((1, out_dim), lambda i: (0, 0)),
            pl.BlockSpec((hc, H), lambda i: (0, 0)),
            pl.BlockSpec((H, hc), lambda i: (0, 0)),
        ],
        out_specs=pl.BlockSpec((B, out_dim), lambda i: (i, 0)),
        out_shape=jax.ShapeDtypeStruct((N, out_dim), jnp.float32),
    )(acc0, acc1, dall, xl, xr, skip, mean_attr, We, att_row, bias, g, bnb,
      S, ST)


def _head(h, w1, b1, w2, b2):
    B = 1000
    f1 = w1.shape[1]

    def kern(h_ref, w1r, b1r, w2r, b2r, o_ref):
        y = jnp.dot(h_ref[...], w1r[...],
                    preferred_element_type=jnp.float32) + b1r[...]
        y = jnp.where(y > 0, y, jnp.exp(y) - 1.0)
        o_ref[...] = jnp.dot(y, w2r[...],
                             preferred_element_type=jnp.float32) + b2r[...]

    return pl.pallas_call(
        kern,
        grid=(N // B,),
        in_specs=[
            pl.BlockSpec((B, HID), lambda i: (i, 0)),
            pl.BlockSpec((HID, f1), lambda i: (0, 0)),
            pl.BlockSpec((1, f1), lambda i: (0, 0)),
            pl.BlockSpec((f1, OUT_CH), lambda i: (0, 0)),
            pl.BlockSpec((1, OUT_CH), lambda i: (0, 0)),
        ],
        out_specs=pl.BlockSpec((B, OUT_CH), lambda i: (i, 0)),
        out_shape=jax.ShapeDtypeStruct((N, OUT_CH), jnp.float32),
    )(h, w1, b1.reshape(1, f1), w2, b2.reshape(1, OUT_CH))


@jax.jit
def kernel(x, edge_index, edge_attr, params):
    src = edge_index[0].astype(jnp.int32)
    dst = edge_index[1].astype(jnp.int32)
    maskf = (src != dst).astype(jnp.float32).reshape(E, 1)

    part = _masked_sum(edge_attr, maskf)
    tot = jnp.sum(part[::8], axis=0)
    mean_attr = (tot[:EDGE_DIM] / tot[EDGE_DIM]).reshape(1, EDGE_DIM)

    h = _mm(x, params["W_in"], params["b_in"], act="elu")

    heads = [HEADS, HEADS, 1]
    concat = [True, True, False]
    for i in range(3):
        p = params["conv%d" % i]
        H = heads[i]
        hc = H * HID
        out_dim = hc if concat[i] else HID

        if hc < 128:
            xlp = _mm(h, jnp.pad(p["Wl"], ((0, 0), (0, 128 - hc))),
                      jnp.pad(p["bl"], (0, 128 - hc)))
            xrp = _mm(h, jnp.pad(p["Wr"], ((0, 0), (0, 128 - hc))),
                      jnp.pad(p["br"], (0, 128 - hc)))
            xl, xr = xlp[:, :hc], xrp[:, :hc]
        else:
            xlp = xl = _mm(h, p["Wl"], p["bl"])
            xrp = xr = _mm(h, p["Wr"], p["br"])
        skip = _mm(h, params["Ws%d" % i], params["bs%d" % i])
        e_all = _mm(edge_attr, p["We"], jnp.zeros((hc,), jnp.float32),
                    block=8000)
        att_flat = p["att"].reshape(hc)
        zeros = jnp.zeros((N, 128), jnp.float32)

        acc, den = _sc_edge_pass(src, dst, xlp, xrp, e_all, att_flat, zeros,
                                 hc, H)
        acc = acc.reshape(NCORE, N, 128)
        den = den.reshape(NCORE, -1, 128)
        dall = den[:, :N // 16, :16 * H].reshape(NCORE, N, H)

        sel = jnp.repeat(jnp.eye(H, dtype=jnp.float32), HID, axis=0)
        h = _combine(acc[0], acc[1], dall, xl, xr, skip, mean_attr, p["We"],
                     att_flat.reshape(1, hc), p["bias"].reshape(1, out_dim),
                     params["g%d" % i].reshape(1, out_dim),
                     params["bn%d" % i].reshape(1, out_dim),
                     sel, sel.T, hc, H, out_dim, do_elu=(i < 2))

    return _head(h, params["Wc1"], params["bc1"], params["Wc2"], params["bc2"])


# speedup vs baseline: 1.1531x; 1.1531x over previous
import functools
import jax
import jax.numpy as jnp
from jax import lax
from jax.experimental import pallas as pl
from jax.experimental.pallas import tpu as pltpu
from jax.experimental.pallas import tpu_sc as plsc

N = 10000
E = 320000
IN_CH = 128
HID = 16
HEADS = 8
OUT_CH = 32
EDGE_DIM = 16
NEG = 0.2

NCORE = 2
NSUB = 16
NWORK = NCORE * NSUB
EPW = E // NWORK
WIN = 40
NWIN = EPW // WIN


def _mm(x, w, b, act=None, block=1000):
    n, fi = x.shape
    fo = w.shape[1]

    def kern(x_ref, w_ref, b_ref, o_ref):
        y = jnp.dot(x_ref[...], w_ref[...],
                    preferred_element_type=jnp.float32) + b_ref[...]
        if act == "elu":
            y = jnp.where(y > 0, y, jnp.exp(y) - 1.0)
        o_ref[...] = y

    return pl.pallas_call(
        kern,
        grid=(n // block,),
        in_specs=[
            pl.BlockSpec((block, fi), lambda i: (i, 0)),
            pl.BlockSpec((fi, fo), lambda i: (0, 0)),
            pl.BlockSpec((1, fo), lambda i: (0, 0)),
        ],
        out_specs=pl.BlockSpec((block, fo), lambda i: (i, 0)),
        out_shape=jax.ShapeDtypeStruct((n, fo), jnp.float32),
    )(x, w, b.reshape(1, fo))


def _masked_sum(edge_attr, maskf):
    BE = 8000
    G = E // BE

    def kern(a_ref, m_ref, o_ref):
        s = jnp.sum(a_ref[...] * m_ref[...], axis=0, keepdims=True)
        c = jnp.sum(m_ref[...]).reshape(1, 1)
        row = jnp.concatenate(
            [s, c, jnp.zeros((1, 128 - EDGE_DIM - 1), jnp.float32)], axis=1)
        o_ref[...] = jnp.concatenate(
            [row, jnp.zeros((7, 128), jnp.float32)], axis=0)

    return pl.pallas_call(
        kern,
        grid=(G,),
        in_specs=[
            pl.BlockSpec((BE, EDGE_DIM), lambda i: (i, 0)),
            pl.BlockSpec((BE, 1), lambda i: (i, 0)),
        ],
        out_specs=pl.BlockSpec((8, 128), lambda i: (i, 0)),
        out_shape=jax.ShapeDtypeStruct((8 * G, 128), jnp.float32),
    )(edge_attr, maskf)


def _sc_edge_pass(src, dst, xl, xr, e_all, att_flat, zeros, hc, H):
    mesh = plsc.VectorSubcoreMesh(core_axis_name="c", subcore_axis_name="s",
                                  num_cores=NCORE, num_subcores=NSUB)
    ND = N // 16
    NDP = ND + 7

    def body(src_hbm, dst_hbm, xl_hbm, xr_hbm, e_hbm, att_hbm, zero_hbm,
             out_hbm, den_hbm,
             srcvA, dstvA, dstpA, dst8vA, glvA, grvA, evA, maskvA,
             semLA, semG1A, semG2A,
             srcvB, dstvB, dstpB, dst8vB, glvB, grvB, evB, maskvB,
             semLB, semG1B, semG2B,
             obuf2, attv, acc_sh, den_sh):
        cid = lax.axis_index("c")
        sid = lax.axis_index("s")
        SETS = (
            (srcvA, dstvA, dstpA, dst8vA, glvA, grvA, evA, maskvA,
             semLA, semG1A, semG2A),
            (srcvB, dstvB, dstpB, dst8vB, glvB, grvB, evB, maskvB,
             semLB, semG1B, semG2B),
        )

        @pl.when(sid < 10)
        def _():
            pltpu.sync_copy(zero_hbm.at[pl.ds(sid * 1000, 1000)],
                            acc_sh.at[pl.ds(sid * 1000, 1000)])

        @pl.when(sid == 0)
        def _():
            pltpu.sync_copy(zero_hbm.at[pl.ds(0, NDP)], den_sh)

        pltpu.sync_copy(att_hbm, attv)
        zero16v = jnp.zeros((16,), jnp.float32)

        def orow(r, c):
            for g in range(8):
                obuf2[r, pl.ds(g * 16, 16)] = zero16v
            return c

        lax.fori_loop(0, WIN, orow, 0, unroll=False)
        plsc.subcore_barrier()

        att_regs = [attv[pl.ds(h * 16, 16)] for h in range(H)]
        lanes = lax.iota(jnp.int32, 16)
        zero16 = jnp.zeros((16,), jnp.float32)
        base0 = (cid * NSUB + sid) * EPW

        gdn = lax.GatherDimensionNumbers(
            offset_dims=(), collapsed_slice_dims=(0,), start_index_map=(0,))
        bfly_idx = [(jnp.arange(16, dtype=jnp.int32) ^ sh)[:, None]
                    for sh in (1, 2, 4, 8)]

        def lanesum(v):
            for idx in bfly_idx:
                v = v + lax.gather(v, idx, gdn, (1,), unique_indices=True,
                                   mode=lax.GatherScatterMode.PROMISE_IN_BOUNDS)
            return v

        rot8_idx = ((jnp.arange(16, dtype=jnp.int32) - 8) & 15)[:, None]

        def rot8(v):
            return lax.gather(v, rot8_idx, gdn, (1,), unique_indices=True,
                              mode=lax.GatherScatterMode.PROMISE_IN_BOUNDS)

        def prefetch(S, base):
            (srcv, dstv, dstp, dst8v, glv, grv, ev, maskv,
             semL, semG1, semG2) = S
            d1 = pltpu.async_copy(src_hbm.at[pl.ds(base, WIN)], srcv, semL)
            d2 = pltpu.async_copy(dst_hbm.at[pl.ds(base, WIN)], dstv, semL)
            d3 = pltpu.async_copy(dst_hbm.at[pl.ds(base, WIN)],
                                  dstp.at[pl.ds(0, WIN)], semL)
            d4 = pltpu.async_copy(e_hbm.at[pl.ds(base, WIN)], ev, semL)
            d1.wait()
            d2.wait()
            d3.wait()
            d4.wait()
            for g0 in (0, 16, WIN - 16):
                sv = srcv[pl.ds(g0, 16)]
                dv = dstv[pl.ds(g0, 16)]
                maskv[pl.ds(g0, 16)] = jnp.where(sv != dv, 1.0, 0.0)
                dst8v[pl.ds(g0, 16)] = lax.shift_right_logical(dv, 4)
            pltpu.async_copy(xl_hbm.at[srcv], glv, semG1)
            pltpu.async_copy(xr_hbm.at[dstv], grv, semG2)

        def do_window(S):
            (srcv, dstv, dstp, dst8v, glv, grv, ev, maskv,
             semL, semG1, semG2) = S
            pltpu.make_async_copy(xl_hbm.at[srcv], glv, semG1).wait()
            pltpu.make_async_copy(xr_hbm.at[dstv], grv, semG2).wait()

            def edge(i, c):
                maskf = maskv[pl.ds(i, 16)][0]
                d_i = dstp[pl.ds(i, 16)][0]
                gls = []
                av = zero16
                for h in range(H):
                    gl_h = glv[i, pl.ds(h * 16, 16)]
                    gr_h = grv[i, pl.ds(h * 16, 16)]
                    e_h = ev[i, pl.ds(h * 16, 16)]
                    m = gl_h + gr_h + e_h
                    m = jnp.maximum(m, NEG * m)
                    av = jnp.where(lanes == h, lanesum(m * att_regs[h]), av)
                    gls.append(gl_h)
                exv = jnp.exp(av) * maskf
                k = d_i & 15
                if H == 8:
                    grp = lax.shift_right_logical(k, 1)
                    hi = (k & 1) == 1
                    row16 = jnp.where(hi,
                                      jnp.where(lanes >= 8, rot8(exv), 0.0),
                                      jnp.where(lanes < 8, exv, 0.0))
                    for g4 in range(8):
                        obuf2[i, pl.ds(g4 * 16, 16)] = jnp.where(
                            grp == g4, row16, 0.0)
                else:
                    obuf2[i, pl.ds(0, 16)] = jnp.where(lanes == k, exv[0],
                                                       0.0)
                for h in range(H):
                    glv[i, pl.ds(h * 16, 16)] = gls[h] * exv[h]
                return c

            lax.fori_loop(0, WIN, edge, 0, unroll=False)
            pltpu.sync_copy(glv, acc_sh.at[dstv], add=True)
            pltpu.sync_copy(obuf2, den_sh.at[dst8v], add=True)

        prefetch(SETS[0], base0)
        prefetch(SETS[1], base0 + WIN)

        def superstep(ks, carry):
            for off in (0, 1):
                S = SETS[off]
                do_window(S)

                @pl.when(ks < NWIN // 2 - 1)
                def _():
                    prefetch(S, base0 + (2 * ks + off + 2) * WIN)
            return carry

        lax.fori_loop(0, NWIN // 2, superstep, 0, unroll=False)

        plsc.subcore_barrier()

        @pl.when(sid < 10)
        def _():
            pltpu.sync_copy(acc_sh.at[pl.ds(sid * 1000, 1000)],
                            out_hbm.at[pl.ds(cid * N + sid * 1000, 1000)])

        @pl.when(sid == 0)
        def _():
            pltpu.sync_copy(den_sh, den_hbm.at[pl.ds(cid * NDP, NDP)])

    kern = pl.kernel(
        body,
        out_type=(
            jax.ShapeDtypeStruct((NCORE * N, 128), jnp.float32),
            jax.ShapeDtypeStruct((NCORE * NDP, 128), jnp.float32),
        ),
        mesh=mesh,
        scratch_types=(
            [
                pltpu.VMEM((WIN,), jnp.int32),
                pltpu.VMEM((WIN,), jnp.int32),
                pltpu.VMEM((WIN + 16,), jnp.int32),
                pltpu.VMEM((WIN,), jnp.int32),
                pltpu.VMEM((WIN, 128), jnp.float32),
                pltpu.VMEM((WIN, 128), jnp.float32),
                pltpu.VMEM((WIN, hc), jnp.float32),
                pltpu.VMEM((WIN + 16,), jnp.float32),
                pltpu.SemaphoreType.DMA,
                pltpu.SemaphoreType.DMA,
                pltpu.SemaphoreType.DMA,
            ] * 2
            + [
                pltpu.VMEM((WIN, 128), jnp.float32),
                pltpu.VMEM((hc,), jnp.float32),
                pltpu.VMEM_SHARED((N, 128), jnp.float32),
                pltpu.VMEM_SHARED((NDP, 128), jnp.float32),
            ]
        ),
    )
    return kern(src, dst, xl, xr, e_all, att_flat, zeros)


def _combine(acc0, acc1, dall, xl, xr, skip, mean_attr, We, att_row, bias,
             g, bnb, S, ST, hc, H, out_dim, do_elu):
    B = 1000

    def kern(a0, a1, dl, xl_r, xr_r, sk, ma, we, at, bi, gg, bb, s_r, st_r,
             o_ref):
        num = (a0[...] + a1[...])[:, :hc]
        den = jnp.sum(dl[...], axis=0)
        e_self = jnp.dot(ma[...], we[...],
                         preferred_element_type=jnp.float32)
        ms = xl_r[...] + xr_r[...] + e_self
        ms = jnp.maximum(ms, NEG * ms)
        alpha_s = jnp.dot(ms * at[...], s_r[...],
                          preferred_element_type=jnp.float32)
        exs = jnp.exp(alpha_s)
        rep = jnp.dot(exs, st_r[...], preferred_element_type=jnp.float32)
        num = num + rep * xl_r[...]
        denf = jnp.dot(den + exs, st_r[...],
                       preferred_element_type=jnp.float32) + 1e-16
        out = num / denf
        out = out + bi[...] + sk[...]
        mu = jnp.mean(out, axis=-1, keepdims=True)
        var = jnp.mean((out - mu) ** 2, axis=-1, keepdims=True)
        out = (out - mu) / jnp.sqrt(var + 1e-5) * gg[...] + bb[...]
        if do_elu:
            out = jnp.where(out > 0, out, jnp.exp(out) - 1.0)
        o_ref[...] = out

    return pl.pallas_call(
        kern,
        grid=(N // B,),
        in_specs=[
            pl.BlockSpec((B, 128), lambda i: (i, 0)),
            pl.BlockSpec((B, 128), lambda i: (i, 0)),
            pl.BlockSpec((NCORE, B, H), lambda i: (0, i, 0)),
            pl.BlockSpec((B, hc), lambda i: (i, 0)),
            pl.BlockSpec((B, hc), lambda i: (i, 0)),
            pl.BlockSpec((B, out_dim), lambda i: (i, 0)),
            pl.BlockSpec((1, EDGE_DIM), lambda i: (0, 0)),
            pl.BlockSpec((EDGE_DIM, hc), lambda i: (0, 0)),
            pl.BlockSpec((1, hc), lambda i: (0, 0)),
            pl.BlockSpec((1, out_dim), lambda i: (0, 0)),
            pl.BlockSpec((1, out_dim), lambda i: (0, 0)),
            pl.BlockSpec((1, out_dim), lambda i: (0, 0)),
            pl.BlockSpec((hc, H), lambda i: (0, 0)),
            pl.BlockSpec((H, hc), lambda i: (0, 0)),
        ],
        out_specs=pl.BlockSpec((B, out_dim), lambda i: (i, 0)),
        out_shape=jax.ShapeDtypeStruct((N, out_dim), jnp.float32),
    )(acc0, acc1, dall, xl, xr, skip, mean_attr, We, att_row, bias, g, bnb,
      S, ST)


def _head(h, w1, b1, w2, b2):
    B = 1000
    f1 = w1.shape[1]

    def kern(h_ref, w1r, b1r, w2r, b2r, o_ref):
        y = jnp.dot(h_ref[...], w1r[...],
                    preferred_element_type=jnp.float32) + b1r[...]
        y = jnp.where(y > 0, y, jnp.exp(y) - 1.0)
        o_ref[...] = jnp.dot(y, w2r[...],
                             preferred_element_type=jnp.float32) + b2r[...]

    return pl.pallas_call(
        kern,
        grid=(N // B,),
        in_specs=[
            pl.BlockSpec((B, HID), lambda i: (i, 0)),
            pl.BlockSpec((HID, f1), lambda i: (0, 0)),
            pl.BlockSpec((1, f1), lambda i: (0, 0)),
            pl.BlockSpec((f1, OUT_CH), lambda i: (0, 0)),
            pl.BlockSpec((1, OUT_CH), lambda i: (0, 0)),
        ],
        out_specs=pl.BlockSpec((B, OUT_CH), lambda i: (i, 0)),
        out_shape=jax.ShapeDtypeStruct((N, OUT_CH), jnp.float32),
    )(h, w1, b1.reshape(1, f1), w2, b2.reshape(1, OUT_CH))


@jax.jit
def kernel(x, edge_index, edge_attr, params):
    src = edge_index[0].astype(jnp.int32)
    dst = edge_index[1].astype(jnp.int32)
    maskf = (src != dst).astype(jnp.float32).reshape(E, 1)

    part = _masked_sum(edge_attr, maskf)
    tot = jnp.sum(part[::8], axis=0)
    mean_attr = (tot[:EDGE_DIM] / tot[EDGE_DIM]).reshape(1, EDGE_DIM)

    h = _mm(x, params["W_in"], params["b_in"], act="elu")

    heads = [HEADS, HEADS, 1]
    concat = [True, True, False]
    for i in range(3):
        p = params["conv%d" % i]
        H = heads[i]
        hc = H * HID
        out_dim = hc if concat[i] else HID

        if hc < 128:
            xlp = _mm(h, jnp.pad(p["Wl"], ((0, 0), (0, 128 - hc))),
                      jnp.pad(p["bl"], (0, 128 - hc)))
            xrp = _mm(h, jnp.pad(p["Wr"], ((0, 0), (0, 128 - hc))),
                      jnp.pad(p["br"], (0, 128 - hc)))
            xl, xr = xlp[:, :hc], xrp[:, :hc]
        else:
            xlp = xl = _mm(h, p["Wl"], p["bl"])
            xrp = xr = _mm(h, p["Wr"], p["br"])
        skip = _mm(h, params["Ws%d" % i], params["bs%d" % i])
        e_all = _mm(edge_attr, p["We"], jnp.zeros((hc,), jnp.float32),
                    block=8000)
        att_flat = p["att"].reshape(hc)
        zeros = jnp.zeros((N, 128), jnp.float32)

        acc, den = _sc_edge_pass(src, dst, xlp, xrp, e_all, att_flat, zeros,
                                 hc, H)
        acc = acc.reshape(NCORE, N, 128)
        den = den.reshape(NCORE, -1, 128)
        dall = den[:, :N // 16, :16 * H].reshape(NCORE, N, H)

        sel = jnp.repeat(jnp.eye(H, dtype=jnp.float32), HID, axis=0)
        h = _combine(acc[0], acc[1], dall, xl, xr, skip, mean_attr, p["We"],
                     att_flat.reshape(1, hc), p["bias"].reshape(1, out_dim),
                     params["g%d" % i].reshape(1, out_dim),
                     params["bn%d" % i].reshape(1, out_dim),
                     sel, sel.T, hc, H, out_dim, do_elu=(i < 2))

    return _head(h, params["Wc1"], params["bc1"], params["Wc2"], params["bc2"])


# speedup vs baseline: 1.3267x; 1.1506x over previous
import functools
import jax
import jax.numpy as jnp
from jax import lax
from jax.experimental import pallas as pl
from jax.experimental.pallas import tpu as pltpu
from jax.experimental.pallas import tpu_sc as plsc

N = 10000
E = 320000
IN_CH = 128
HID = 16
HEADS = 8
OUT_CH = 32
EDGE_DIM = 16
NEG = 0.2

NCORE = 2
NSUB = 16
NWORK = NCORE * NSUB
EPW = E // NWORK
WIN = 40
NWIN = EPW // WIN


def _mm(x, w, b, act=None, block=1000):
    n, fi = x.shape
    fo = w.shape[1]

    def kern(x_ref, w_ref, b_ref, o_ref):
        y = jnp.dot(x_ref[...], w_ref[...],
                    preferred_element_type=jnp.float32) + b_ref[...]
        if act == "elu":
            y = jnp.where(y > 0, y, jnp.exp(y) - 1.0)
        o_ref[...] = y

    return pl.pallas_call(
        kern,
        grid=(n // block,),
        in_specs=[
            pl.BlockSpec((block, fi), lambda i: (i, 0)),
            pl.BlockSpec((fi, fo), lambda i: (0, 0)),
            pl.BlockSpec((1, fo), lambda i: (0, 0)),
        ],
        out_specs=pl.BlockSpec((block, fo), lambda i: (i, 0)),
        out_shape=jax.ShapeDtypeStruct((n, fo), jnp.float32),
    )(x, w, b.reshape(1, fo))


def _masked_sum(edge_attr, maskf):
    BE = 8000
    G = E // BE

    def kern(a_ref, m_ref, o_ref):
        s = jnp.sum(a_ref[...] * m_ref[...], axis=0, keepdims=True)
        c = jnp.sum(m_ref[...]).reshape(1, 1)
        row = jnp.concatenate(
            [s, c, jnp.zeros((1, 128 - EDGE_DIM - 1), jnp.float32)], axis=1)
        o_ref[...] = jnp.concatenate(
            [row, jnp.zeros((7, 128), jnp.float32)], axis=0)

    return pl.pallas_call(
        kern,
        grid=(G,),
        in_specs=[
            pl.BlockSpec((BE, EDGE_DIM), lambda i: (i, 0)),
            pl.BlockSpec((BE, 1), lambda i: (i, 0)),
        ],
        out_specs=pl.BlockSpec((8, 128), lambda i: (i, 0)),
        out_shape=jax.ShapeDtypeStruct((8 * G, 128), jnp.float32),
    )(edge_attr, maskf)


def _sc_edge_pass(src, dst, xl, xr, e_all, att_flat, zeros, hc, H):
    mesh = plsc.VectorSubcoreMesh(core_axis_name="c", subcore_axis_name="s",
                                  num_cores=NCORE, num_subcores=NSUB)
    ND = N // 16
    NDP = ND + 7

    def body(src_hbm, dst_hbm, xl_hbm, xr_hbm, e_hbm, att_hbm, zero_hbm,
             out_hbm, den_hbm,
             srcvA, dstvA, dstpA, dst8vA, glvA, grvA, evA, maskvA,
             semLA, semG1A, semG2A,
             srcvB, dstvB, dstpB, dst8vB, glvB, grvB, evB, maskvB,
             semLB, semG1B, semG2B,
             obuf, obuf2, attv, acc_sh, den_sh):
        cid = lax.axis_index("c")
        sid = lax.axis_index("s")
        SETS = (
            (srcvA, dstvA, dstpA, dst8vA, glvA, grvA, evA, maskvA,
             semLA, semG1A, semG2A),
            (srcvB, dstvB, dstpB, dst8vB, glvB, grvB, evB, maskvB,
             semLB, semG1B, semG2B),
        )

        @pl.when(sid < 10)
        def _():
            pltpu.sync_copy(zero_hbm.at[pl.ds(sid * 1000, 1000)],
                            acc_sh.at[pl.ds(sid * 1000, 1000)])

        @pl.when(sid == 0)
        def _():
            pltpu.sync_copy(zero_hbm.at[pl.ds(0, NDP)], den_sh)

        pltpu.sync_copy(att_hbm, attv)
        zero16v = jnp.zeros((16,), jnp.float32)

        def orow(r, c):
            for g in range(8):
                obuf[r, pl.ds(g * 16, 16)] = zero16v
                obuf2[r, pl.ds(g * 16, 16)] = zero16v
            return c

        lax.fori_loop(0, WIN, orow, 0, unroll=False)
        plsc.subcore_barrier()

        att_regs = [attv[pl.ds(h * 16, 16)] for h in range(H)]
        lanes = lax.iota(jnp.int32, 16)
        zero16 = jnp.zeros((16,), jnp.float32)
        base0 = (cid * NSUB + sid) * EPW

        gdn = lax.GatherDimensionNumbers(
            offset_dims=(), collapsed_slice_dims=(0,), start_index_map=(0,))
        bfly_idx = [(jnp.arange(16, dtype=jnp.int32) ^ sh)[:, None]
                    for sh in (1, 2, 4, 8)]

        def lanesum(v):
            for idx in bfly_idx:
                v = v + lax.gather(v, idx, gdn, (1,), unique_indices=True,
                                   mode=lax.GatherScatterMode.PROMISE_IN_BOUNDS)
            return v

        rot8_idx = ((jnp.arange(16, dtype=jnp.int32) - 8) & 15)[:, None]

        def rot8(v):
            return lax.gather(v, rot8_idx, gdn, (1,), unique_indices=True,
                              mode=lax.GatherScatterMode.PROMISE_IN_BOUNDS)

        def prefetch(S, base):
            (srcv, dstv, dstp, dst8v, glv, grv, ev, maskv,
             semL, semG1, semG2) = S
            d1 = pltpu.async_copy(src_hbm.at[pl.ds(base, WIN)], srcv, semL)
            d2 = pltpu.async_copy(dst_hbm.at[pl.ds(base, WIN)], dstv, semL)
            d3 = pltpu.async_copy(dst_hbm.at[pl.ds(base, WIN)],
                                  dstp.at[pl.ds(0, WIN)], semL)
            d4 = pltpu.async_copy(e_hbm.at[pl.ds(base, WIN)], ev, semL)
            d1.wait()
            d2.wait()
            d3.wait()
            d4.wait()
            for g0 in (0, 16, WIN - 16):
                sv = srcv[pl.ds(g0, 16)]
                dv = dstv[pl.ds(g0, 16)]
                maskv[pl.ds(g0, 16)] = jnp.where(sv != dv, 1.0, 0.0)
                dst8v[pl.ds(g0, 16)] = lax.shift_right_logical(dv, 4)
            pltpu.async_copy(xl_hbm.at[srcv], glv, semG1)
            pltpu.async_copy(xr_hbm.at[dstv], grv, semG2)

        def do_window(S):
            (srcv, dstv, dstp, dst8v, glv, grv, ev, maskv,
             semL, semG1, semG2) = S
            pltpu.make_async_copy(xl_hbm.at[srcv], glv, semG1).wait()
            pltpu.make_async_copy(xr_hbm.at[dstv], grv, semG2).wait()

            def edge(i, c):
                maskf = maskv[pl.ds(i, 16)][0]
                d_i = dstp[pl.ds(i, 16)][0]
                gls = []
                av = zero16
                for h in range(H):
                    gl_h = glv[i, pl.ds(h * 16, 16)]
                    gr_h = grv[i, pl.ds(h * 16, 16)]
                    e_h = ev[i, pl.ds(h * 16, 16)]
                    m = gl_h + gr_h + e_h
                    m = jnp.maximum(m, NEG * m)
                    av = jnp.where(lanes == h, lanesum(m * att_regs[h]), av)
                    gls.append(gl_h)
                exv = jnp.exp(av) * maskf
                k = d_i & 15
                if H == 8:
                    grp = lax.shift_right_logical(k, 1)
                    hi = (k & 1) == 1
                    row16 = jnp.where(hi,
                                      jnp.where(lanes >= 8, rot8(exv), 0.0),
                                      jnp.where(lanes < 8, exv, 0.0))
                    for g4 in range(8):
                        obuf2[i, pl.ds(g4 * 16, 16)] = jnp.where(
                            grp == g4, row16, 0.0)
                else:
                    obuf2[i, pl.ds(0, 16)] = jnp.where(lanes == k, exv[0],
                                                       0.0)
                for h in range(H):
                    obuf[i, pl.ds(h * 16, 16)] = gls[h] * exv[h]
                return c

            lax.fori_loop(0, WIN, edge, 0, unroll=False)
            pltpu.sync_copy(obuf, acc_sh.at[dstv], add=True)
            pltpu.sync_copy(obuf2, den_sh.at[dst8v], add=True)

        prefetch(SETS[0], base0)
        prefetch(SETS[1], base0 + WIN)

        def superstep(ks, carry):
            for off in (0, 1):
                S = SETS[off]
                do_window(S)

                @pl.when(ks < NWIN // 2 - 1)
                def _():
                    prefetch(S, base0 + (2 * ks + off + 2) * WIN)
            return carry

        lax.fori_loop(0, NWIN // 2, superstep, 0, unroll=False)

        plsc.subcore_barrier()

        @pl.when(sid < 10)
        def _():
            pltpu.sync_copy(acc_sh.at[pl.ds(sid * 1000, 1000)],
                            out_hbm.at[pl.ds(cid * N + sid * 1000, 1000)])

        @pl.when(sid == 0)
        def _():
            pltpu.sync_copy(den_sh, den_hbm.at[pl.ds(cid * NDP, NDP)])

    kern = pl.kernel(
        body,
        out_type=(
            jax.ShapeDtypeStruct((NCORE * N, 128), jnp.float32),
            jax.ShapeDtypeStruct((NCORE * NDP, 128), jnp.float32),
        ),
        mesh=mesh,
        scratch_types=(
            [
                pltpu.VMEM((WIN,), jnp.int32),
                pltpu.VMEM((WIN,), jnp.int32),
                pltpu.VMEM((WIN + 16,), jnp.int32),
                pltpu.VMEM((WIN,), jnp.int32),
                pltpu.VMEM((WIN, 128), jnp.float32),
                pltpu.VMEM((WIN, 128), jnp.float32),
                pltpu.VMEM((WIN, hc), jnp.float32),
                pltpu.VMEM((WIN + 16,), jnp.float32),
                pltpu.SemaphoreType.DMA,
                pltpu.SemaphoreType.DMA,
                pltpu.SemaphoreType.DMA,
            ] * 2
            + [
                pltpu.VMEM((WIN, 128), jnp.float32),
                pltpu.VMEM((WIN, 128), jnp.float32),
                pltpu.VMEM((hc,), jnp.float32),
                pltpu.VMEM_SHARED((N, 128), jnp.float32),
                pltpu.VMEM_SHARED((NDP, 128), jnp.float32),
            ]
        ),
    )
    return kern(src, dst, xl, xr, e_all, att_flat, zeros)


def _combine(acc0, acc1, dall, xl, xr, skip, mean_attr, We, att_row, bias,
             g, bnb, S, ST, hc, H, out_dim, do_elu):
    B = 1000

    def kern(a0, a1, dl, xl_r, xr_r, sk, ma, we, at, bi, gg, bb, s_r, st_r,
             o_ref):
        num = (a0[...] + a1[...])[:, :hc]
        den = jnp.sum(dl[...], axis=0)
        e_self = jnp.dot(ma[...], we[...],
                         preferred_element_type=jnp.float32)
        ms = xl_r[...] + xr_r[...] + e_self
        ms = jnp.maximum(ms, NEG * ms)
        alpha_s = jnp.dot(ms * at[...], s_r[...],
                          preferred_element_type=jnp.float32)
        exs = jnp.exp(alpha_s)
        rep = jnp.dot(exs, st_r[...], preferred_element_type=jnp.float32)
        num = num + rep * xl_r[...]
        denf = jnp.dot(den + exs, st_r[...],
                       preferred_element_type=jnp.float32) + 1e-16
        out = num / denf
        out = out + bi[...] + sk[...]
        mu = jnp.mean(out, axis=-1, keepdims=True)
        var = jnp.mean((out - mu) ** 2, axis=-1, keepdims=True)
        out = (out - mu) / jnp.sqrt(var + 1e-5) * gg[...] + bb[...]
        if do_elu:
            out = jnp.where(out > 0, out, jnp.exp(out) - 1.0)
        o_ref[...] = out

    return pl.pallas_call(
        kern,
        grid=(N // B,),
        in_specs=[
            pl.BlockSpec((B, 128), lambda i: (i, 0)),
            pl.BlockSpec((B, 128), lambda i: (i, 0)),
            pl.BlockSpec((NCORE, B, H), lambda i: (0, i, 0)),
            pl.BlockSpec((B, hc), lambda i: (i, 0)),
            pl.BlockSpec((B, hc), lambda i: (i, 0)),
            pl.BlockSpec((B, out_dim), lambda i: (i, 0)),
            pl.BlockSpec((1, EDGE_DIM), lambda i: (0, 0)),
            pl.BlockSpec((EDGE_DIM, hc), lambda i: (0, 0)),
            pl.BlockSpec((1, hc), lambda i: (0, 0)),
            pl.BlockSpec((1, out_dim), lambda i: (0, 0)),
            pl.BlockSpec((1, out_dim), lambda i: (0, 0)),
            pl.BlockSpec((1, out_dim), lambda i: (0, 0)),
            pl.BlockSpec((hc, H), lambda i: (0, 0)),
            pl.BlockSpec((H, hc), lambda i: (0, 0)),
        ],
        out_specs=pl.BlockSpec((B, out_dim), lambda i: (i, 0)),
        out_shape=jax.ShapeDtypeStruct((N, out_dim), jnp.float32),
    )(acc0, acc1, dall, xl, xr, skip, mean_attr, We, att_row, bias, g, bnb,
      S, ST)


def _head(h, w1, b1, w2, b2):
    B = 1000
    f1 = w1.shape[1]

    def kern(h_ref, w1r, b1r, w2r, b2r, o_ref):
        y = jnp.dot(h_ref[...], w1r[...],
                    preferred_element_type=jnp.float32) + b1r[...]
        y = jnp.where(y > 0, y, jnp.exp(y) - 1.0)
        o_ref[...] = jnp.dot(y, w2r[...],
                             preferred_element_type=jnp.float32) + b2r[...]

    return pl.pallas_call(
        kern,
        grid=(N // B,),
        in_specs=[
            pl.BlockSpec((B, HID), lambda i: (i, 0)),
            pl.BlockSpec((HID, f1), lambda i: (0, 0)),
            pl.BlockSpec((1, f1), lambda i: (0, 0)),
            pl.BlockSpec((f1, OUT_CH), lambda i: (0, 0)),
            pl.BlockSpec((1, OUT_CH), lambda i: (0, 0)),
        ],
        out_specs=pl.BlockSpec((B, OUT_CH), lambda i: (i, 0)),
        out_shape=jax.ShapeDtypeStruct((N, OUT_CH), jnp.float32),
    )(h, w1, b1.reshape(1, f1), w2, b2.reshape(1, OUT_CH))


@jax.jit
def kernel(x, edge_index, edge_attr, params):
    src = edge_index[0].astype(jnp.int32)
    dst = edge_index[1].astype(jnp.int32)
    maskf = (src != dst).astype(jnp.float32).reshape(E, 1)

    part = _masked_sum(edge_attr, maskf)
    tot = jnp.sum(part[::8], axis=0)
    mean_attr = (tot[:EDGE_DIM] / tot[EDGE_DIM]).reshape(1, EDGE_DIM)

    h = _mm(x, params["W_in"], params["b_in"], act="elu")

    heads = [HEADS, HEADS, 1]
    concat = [True, True, False]
    for i in range(3):
        p = params["conv%d" % i]
        H = heads[i]
        hc = H * HID
        out_dim = hc if concat[i] else HID

        if hc < 128:
            xlp = _mm(h, jnp.pad(p["Wl"], ((0, 0), (0, 128 - hc))),
                      jnp.pad(p["bl"], (0, 128 - hc)))
            xrp = _mm(h, jnp.pad(p["Wr"], ((0, 0), (0, 128 - hc))),
                      jnp.pad(p["br"], (0, 128 - hc)))
            xl, xr = xlp[:, :hc], xrp[:, :hc]
        else:
            xlp = xl = _mm(h, p["Wl"], p["bl"])
            xrp = xr = _mm(h, p["Wr"], p["br"])
        skip = _mm(h, params["Ws%d" % i], params["bs%d" % i])
        e_all = _mm(edge_attr, p["We"], jnp.zeros((hc,), jnp.float32),
                    block=8000)
        att_flat = p["att"].reshape(hc)
        zeros = jnp.zeros((N, 128), jnp.float32)

        acc, den = _sc_edge_pass(src, dst, xlp, xrp, e_all, att_flat, zeros,
                                 hc, H)
        acc = acc.reshape(NCORE, N, 128)
        den = den.reshape(NCORE, -1, 128)
        dall = den[:, :N // 16, :16 * H].reshape(NCORE, N, H)

        sel = jnp.repeat(jnp.eye(H, dtype=jnp.float32), HID, axis=0)
        h = _combine(acc[0], acc[1], dall, xl, xr, skip, mean_attr, p["We"],
                     att_flat.reshape(1, hc), p["bias"].reshape(1, out_dim),
                     params["g%d" % i].reshape(1, out_dim),
                     params["bn%d" % i].reshape(1, out_dim),
                     sel, sel.T, hc, H, out_dim, do_elu=(i < 2))

    return _head(h, params["Wc1"], params["bc1"], params["Wc2"], params["bc2"])
